# baseline (speedup 1.0000x reference)
import functools

import jax
import jax.numpy as jnp
from jax import lax
from jax.experimental import pallas as pl
from jax.experimental.pallas import tpu as pltpu
from jax.experimental.pallas import tpu_sc as plsc

_B = 16384
_W = 1000
_KEEP = 100
_RPAD = 104
_ZROWS = 128
_BUFR = _RPAD + _ZROWS
_NC = 2
_NS = 16
_NW = _NC * _NS
_LPW = _B // _NW


def _sc_mask_t(x_hbm, out_hbm, buf, in_sem, out_sem):
    wid = lax.axis_index("s") * _NC + lax.axis_index("c")
    col0 = wid * _LPW

    gather = pltpu.make_async_copy(
        x_hbm.at[pl.ds(0, _RPAD), pl.ds(col0, _LPW)],
        buf.at[pl.ds(0, _RPAD)],
        in_sem,
    )
    gather.start()

    zero = jnp.zeros((16,), jnp.float32)

    def _zero_row(r, carry):
        for off in range(0, _LPW, 16):
            buf[r, pl.ds(off, 16)] = zero
        return carry

    lax.fori_loop(_RPAD, _BUFR, _zero_row, 0)
    gather.wait()
    lax.fori_loop(_KEEP, _RPAD, _zero_row, 0)

    def out_copy(dst_row, src_row, rows):
        return pltpu.make_async_copy(
            buf.at[pl.ds(src_row, rows)],
            out_hbm.at[pl.ds(dst_row, rows), pl.ds(col0, _LPW)],
            out_sem,
        )

    copies = [out_copy(0, 0, _BUFR)]
    for k in range(6):
        copies.append(out_copy(_BUFR + k * _ZROWS, _RPAD, _ZROWS))
    for c in copies:
        c.start()
    for c in copies:
        c.wait()


@functools.cache
def _build():
    mesh = plsc.VectorSubcoreMesh(core_axis_name="c", subcore_axis_name="s")
    return pl.kernel(
        _sc_mask_t,
        mesh=mesh,
        out_type=jax.ShapeDtypeStruct((_W, _B), jnp.float32),
        scratch_types=[
            pltpu.VMEM((_BUFR, _LPW), jnp.float32),
            pltpu.SemaphoreType.DMA,
            pltpu.SemaphoreType.DMA,
        ],
    )


def kernel(sender_input, labels):
    del labels
    return _build()(sender_input.T).T
